# Initial kernel scaffold; baseline (speedup 1.0000x reference)
import jax
import jax.numpy as jnp
from jax.experimental import pallas as pl


def kernel(xyz, pcd, feat):
    raise NotImplementedError("write your pallas kernel here")


# speedup vs baseline: 37.9149x; 37.9149x over previous
import functools

import jax
import jax.numpy as jnp
from jax.experimental import pallas as pl

K = 10
RADIUS = 0.1
MULTIRES = 10
BIG = 1e30
NCAND = 16


def _body(xyz_ref, pcdc_ref, featc_ref, out_ref, *, n_chunks, chunk):
    x = xyz_ref[0]
    pt = x.shape[0]
    r2 = jnp.float32(RADIUS * RADIUS)

    def d2_chunk(c):
        acc = None
        for i in range(3):
            diff = x[:, i : i + 1] - pcdc_ref[0, c, i, :][None, :]
            sq = diff * diff
            acc = sq if acc is None else acc + sq
        return acc

    def extract(buf, count):
        vals = []
        for _ in range(count):
            m = jnp.min(buf, axis=-1, keepdims=True)
            vals.append(m)
            buf = jnp.where(buf <= m, BIG, buf)
        return vals

    def p1_body(c, cand):
        d2 = d2_chunk(c)
        masked = jnp.where(d2 <= r2, d2, BIG)
        vals = extract(masked, K + 1)
        comb = jnp.concatenate(vals + [cand], axis=-1)
        newvals = extract(comb, K + 1)
        pad = jnp.full((pt, NCAND - (K + 1)), BIG, jnp.float32)
        return jnp.concatenate(newvals + [pad], axis=-1)

    cand0 = jnp.full((pt, NCAND), BIG, jnp.float32)
    cand = jax.lax.fori_loop(0, n_chunks, p1_body, cand0)

    final = extract(cand, K + 1)
    t10, t11 = final[K - 1], final[K]
    tau = jnp.minimum(t10 + (t11 - t10) * 0.5, r2)

    nfeat = featc_ref.shape[3]

    def p2_body(c, carry):
        acc_f, cnt = carry
        d2 = d2_chunk(c)
        mask = jnp.where(d2 <= tau, 1.0, 0.0).astype(jnp.float32)
        fc = featc_ref[0, c]
        acc_f = acc_f + jax.lax.dot_general(
            mask, fc, (((1,), (0,)), ((), ())),
            preferred_element_type=jnp.float32)
        cnt = cnt + jnp.sum(mask, axis=-1, keepdims=True)
        return acc_f, cnt

    acc0 = (jnp.zeros((pt, nfeat), jnp.float32), jnp.zeros((pt, 1), jnp.float32))
    acc_f, cnt = jax.lax.fori_loop(0, n_chunks, p2_body, acc0)
    fcd = acc_f / jnp.maximum(cnt, 1.0)

    freqs = jnp.exp2(jax.lax.broadcasted_iota(
        jnp.int32, (1, MULTIRES), 1).astype(jnp.float32))
    parts = [fcd, x]
    for i in range(3):
        xb = x[:, i : i + 1] * freqs
        parts.append(jnp.sin(xb))
        parts.append(jnp.cos(xb))
    out_ref[0] = jnp.concatenate(parts, axis=-1)


def kernel(xyz, pcd, feat):
    b, p, _ = xyz.shape
    n = pcd.shape[1]
    c = feat.shape[2]
    pt = min(512, p)
    chunk = min(2048, n)
    n_chunks = n // chunk
    out_dim = c + 3 + 3 * 2 * MULTIRES
    pcd_c = jnp.swapaxes(pcd, 1, 2).reshape(b, 3, n_chunks, chunk)
    pcd_c = jnp.swapaxes(pcd_c, 1, 2)
    feat_c = feat.reshape(b, n_chunks, chunk, c)
    return pl.pallas_call(
        functools.partial(_body, n_chunks=n_chunks, chunk=chunk),
        grid=(b, p // pt),
        in_specs=[
            pl.BlockSpec((1, pt, 3), lambda bi, pi: (bi, pi, 0)),
            pl.BlockSpec((1, n_chunks, 3, chunk), lambda bi, pi: (bi, 0, 0, 0)),
            pl.BlockSpec((1, n_chunks, chunk, c), lambda bi, pi: (bi, 0, 0, 0)),
        ],
        out_specs=pl.BlockSpec((1, pt, out_dim), lambda bi, pi: (bi, pi, 0)),
        out_shape=jax.ShapeDtypeStruct((b, p, out_dim), jnp.float32),
    )(xyz, pcd_c, feat_c)


# speedup vs baseline: 40.1537x; 1.0590x over previous
import functools

import jax
import jax.numpy as jnp
from jax.experimental import pallas as pl

K = 10
RADIUS = 0.1
MULTIRES = 10
BIG = 1e30
NCAND = 16


def _body(xyz_ref, pcdc_ref, featc_ref, out_ref, *, n_chunks, chunk):
    x = xyz_ref[0]
    pt = x.shape[0]
    r2 = jnp.float32(RADIUS * RADIUS)

    def d2_chunk(c):
        acc = None
        for i in range(3):
            diff = x[:, i : i + 1] - pcdc_ref[0, c, i, :][None, :]
            sq = diff * diff
            acc = sq if acc is None else acc + sq
        return acc

    def extract(buf, count):
        vals = []
        m = None
        for _ in range(count):
            filt = buf if m is None else jnp.where(buf > m, buf, BIG)
            m = jnp.min(filt, axis=-1, keepdims=True)
            vals.append(m)
        return vals

    def p1_body(c, cand):
        d2 = d2_chunk(c)
        masked = jnp.where(d2 <= r2, d2, BIG)
        vals = extract(masked, K + 1)
        comb = jnp.concatenate(vals + [cand], axis=-1)
        newvals = extract(comb, K + 1)
        pad = jnp.full((pt, NCAND - (K + 1)), BIG, jnp.float32)
        return jnp.concatenate(newvals + [pad], axis=-1)

    cand0 = jnp.full((pt, NCAND), BIG, jnp.float32)
    cand = jax.lax.fori_loop(0, n_chunks, p1_body, cand0)

    final = extract(cand, K + 1)
    t10, t11 = final[K - 1], final[K]
    tau = jnp.minimum(t10 + (t11 - t10) * 0.5, r2)

    nfeat = featc_ref.shape[3]

    def p2_body(c, acc_f):
        d2 = d2_chunk(c)
        mask = jnp.where(d2 <= tau, 1.0, 0.0).astype(jnp.float32)
        fc = featc_ref[0, c]
        return acc_f + jax.lax.dot_general(
            mask, fc, (((1,), (0,)), ((), ())),
            preferred_element_type=jnp.float32)

    acc0 = jnp.zeros((pt, nfeat), jnp.float32)
    acc_f = jax.lax.fori_loop(0, n_chunks, p2_body, acc0)
    cnt = acc_f[:, nfeat - 1 : nfeat]
    fcd = acc_f[:, : nfeat - 1] / jnp.maximum(cnt, 1.0)

    freqs = jnp.exp2(jax.lax.broadcasted_iota(
        jnp.int32, (1, MULTIRES), 1).astype(jnp.float32))
    parts = [fcd, x]
    for i in range(3):
        xb = x[:, i : i + 1] * freqs
        parts.append(jnp.sin(xb))
        parts.append(jnp.cos(xb))
    out_ref[0] = jnp.concatenate(parts, axis=-1)


def kernel(xyz, pcd, feat):
    b, p, _ = xyz.shape
    n = pcd.shape[1]
    c = feat.shape[2]
    pt = min(512, p)
    chunk = min(2048, n)
    n_chunks = n // chunk
    out_dim = c + 3 + 3 * 2 * MULTIRES
    pcd_c = jnp.swapaxes(pcd, 1, 2).reshape(b, 3, n_chunks, chunk)
    pcd_c = jnp.swapaxes(pcd_c, 1, 2)
    feat_aug = jnp.concatenate(
        [feat, jnp.ones((b, n, 1), jnp.float32)], axis=-1)
    feat_c = feat_aug.reshape(b, n_chunks, chunk, c + 1)
    return pl.pallas_call(
        functools.partial(_body, n_chunks=n_chunks, chunk=chunk),
        grid=(b, p // pt),
        in_specs=[
            pl.BlockSpec((1, pt, 3), lambda bi, pi: (bi, pi, 0)),
            pl.BlockSpec((1, n_chunks, 3, chunk), lambda bi, pi: (bi, 0, 0, 0)),
            pl.BlockSpec((1, n_chunks, chunk, c + 1),
                         lambda bi, pi: (bi, 0, 0, 0)),
        ],
        out_specs=pl.BlockSpec((1, pt, out_dim), lambda bi, pi: (bi, pi, 0)),
        out_shape=jax.ShapeDtypeStruct((b, p, out_dim), jnp.float32),
    )(xyz, pcd_c, feat_c)


# speedup vs baseline: 40.9109x; 1.0189x over previous
import functools

import jax
import jax.numpy as jnp
from jax.experimental import pallas as pl

K = 10
RADIUS = 0.1
MULTIRES = 10
BIG = 1e30
NSUB = 4


def _body(xyz_ref, pcdc_ref, featc_ref, out_ref, *, n_chunks, chunk):
    x = xyz_ref[0]
    pt = x.shape[0]
    sub = chunk // NSUB
    r2 = jnp.float32(RADIUS * RADIUS)

    def d2_sub(c, q):
        acc = None
        for i in range(3):
            diff = x[:, i : i + 1] - pcdc_ref[0, c, q, i, :][None, :]
            sq = diff * diff
            acc = sq if acc is None else acc + sq
        return acc

    def cmpex(a, b):
        return jnp.minimum(a, b), jnp.maximum(a, b)

    def extract(buf, count):
        vals = []
        m = None
        for _ in range(count):
            filt = buf if m is None else jnp.where(buf > m, buf, BIG)
            m = jnp.min(filt, axis=-1, keepdims=True)
            vals.append(m)
        return vals

    def p1_body(c, cand):
        v = [jnp.where(d <= r2, d, BIG)
             for d in (d2_sub(c, q) for q in range(NSUB))]
        v[0], v[1] = cmpex(v[0], v[1])
        v[2], v[3] = cmpex(v[2], v[3])
        v[0], v[2] = cmpex(v[0], v[2])
        v[1], v[3] = cmpex(v[1], v[3])
        v[1], v[2] = cmpex(v[1], v[2])
        v0, v1, v2, v3 = v
        vals = []
        for _ in range(K + 1):
            m = jnp.min(v0, axis=-1, keepdims=True)
            vals.append(m)
            hit = v0 <= m
            v0 = jnp.where(hit, v1, v0)
            v1 = jnp.where(hit, v2, v1)
            v2 = jnp.where(hit, v3, v2)
            v3 = jnp.where(hit, BIG, v3)
        comb = jnp.concatenate(vals + [cand], axis=-1)
        newvals = extract(comb, K + 1)
        pad = jnp.full((pt, 5), BIG, jnp.float32)
        return jnp.concatenate(newvals + [pad], axis=-1)

    cand0 = jnp.full((pt, K + 6), BIG, jnp.float32)
    cand = jax.lax.fori_loop(0, n_chunks, p1_body, cand0)

    final = extract(cand, K + 1)
    t10, t11 = final[K - 1], final[K]
    tau = jnp.minimum(t10 + (t11 - t10) * 0.5, r2)

    nfeat = featc_ref.shape[4]
    def p2_body(c, acc_f):
        for q in range(NSUB):
            mask = jnp.where(d2_sub(c, q) <= tau, 1.0, 0.0)
            fc = featc_ref[0, c, q]
            acc_f = acc_f + jax.lax.dot_general(
                mask.astype(jnp.float32), fc, (((1,), (0,)), ((), ())),
                preferred_element_type=jnp.float32)
        return acc_f

    acc_f = jax.lax.fori_loop(
        0, n_chunks, p2_body, jnp.zeros((pt, nfeat), jnp.float32))
    cnt = acc_f[:, nfeat - 1 : nfeat]
    fcd = acc_f[:, : nfeat - 1] / jnp.maximum(cnt, 1.0)

    out_dim = out_ref.shape[2]
    nf = nfeat - 1
    li = jax.lax.broadcasted_iota(jnp.int32, (1, out_dim), 1)
    x0 = jnp.broadcast_to(x[:, 0:1], (pt, out_dim))
    x1 = jnp.broadcast_to(x[:, 1:2], (pt, out_dim))
    x2 = jnp.broadcast_to(x[:, 2:3], (pt, out_dim))
    b0, b1 = nf + 3, nf + 3 + 2 * MULTIRES
    b2 = b1 + 2 * MULTIRES
    xs = jnp.where(li < b1, x0, jnp.where(li < b2, x1, x2))
    xs = jnp.where(li == nf + 1, x1, xs)
    xs = jnp.where(li == nf + 2, x2, xs)
    blk = jnp.where(li < b1, li - b0, jnp.where(li < b2, li - b1, li - b2))
    is_cos = blk >= MULTIRES
    e = jnp.where(is_cos, blk - MULTIRES, blk)
    ftab = jnp.exp2(jnp.where(e < 0, 0, e).astype(jnp.float32))
    arg = xs * ftab
    pe = jnp.where(is_cos, jnp.cos(arg), jnp.sin(arg))
    pe = jnp.where(li < b0, xs, pe)
    fcd_pad = jnp.pad(fcd, ((0, 0), (0, out_dim - nf)))
    out_ref[0] = jnp.where(li < nf, fcd_pad, pe)


def kernel(xyz, pcd, feat):
    b, p, _ = xyz.shape
    n = pcd.shape[1]
    c = feat.shape[2]
    pt = min(512, p)
    chunk = min(2048, n)
    n_chunks = n // chunk
    sub = chunk // NSUB
    out_dim = c + 3 + 3 * 2 * MULTIRES
    pcd_c = jnp.swapaxes(pcd, 1, 2).reshape(b, 3, n_chunks, NSUB, sub)
    pcd_c = jnp.transpose(pcd_c, (0, 2, 3, 1, 4))
    feat_aug = jnp.concatenate(
        [feat, jnp.ones((b, n, 1), jnp.float32)], axis=-1)
    feat_c = feat_aug.reshape(b, n_chunks, NSUB, sub, c + 1)
    return pl.pallas_call(
        functools.partial(_body, n_chunks=n_chunks, chunk=chunk),
        grid=(b, p // pt),
        in_specs=[
            pl.BlockSpec((1, pt, 3), lambda bi, pi: (bi, pi, 0)),
            pl.BlockSpec((1, n_chunks, NSUB, 3, sub),
                         lambda bi, pi: (bi, 0, 0, 0, 0)),
            pl.BlockSpec((1, n_chunks, NSUB, sub, c + 1),
                         lambda bi, pi: (bi, 0, 0, 0, 0)),
        ],
        out_specs=pl.BlockSpec((1, pt, out_dim), lambda bi, pi: (bi, pi, 0)),
        out_shape=jax.ShapeDtypeStruct((b, p, out_dim), jnp.float32),
    )(xyz, pcd_c, feat_c)
